# baseline (device time: 508134 ns/iter reference)
import jax
import jax.numpy as jnp
from jax import lax
from jax.experimental import pallas as pl
from jax.experimental.pallas import tpu as pltpu

C = 16
LAG = 4


def kernel(x, dy):
    bl, d = x.shape
    _, f = dy.shape
    blk = d // 4
    cw = f // C

    xi = lax.axis_index("x")
    yi = lax.axis_index("y")
    zi = lax.axis_index("z")
    r = 2 * yi + zi

    x_slice = lax.dynamic_slice(x, (0, r * blk), (bl, blk))
    xt = x_slice.T.astype(jnp.bfloat16)

    def body(xs_ref, dy_ref, out_ref, xs, dyb, dybf, pb, vb, xrv,
             x_send, x_recv, y_send, y_recv, z_send, z_recv,
             x_credit, copy_a, copy_b, copy_dy):
        xi = lax.axis_index("x")
        yi = lax.axis_index("y")
        zi = lax.axis_index("z")
        is_builder = yi == xi

        barrier = pltpu.get_barrier_semaphore()
        for nbr in ((1 - xi, yi, zi), (xi, 1 - yi, zi), (xi, yi, 1 - zi)):
            pl.semaphore_signal(
                barrier, inc=1, device_id=nbr,
                device_id_type=pl.DeviceIdType.MESH,
            )
        pl.semaphore_wait(barrier, 3)

        mine = out_ref.at[pl.ds(zi * blk, blk)]
        stage = out_ref.at[pl.ds((1 - zi) * blk, blk)]

        def cols(c):
            return pl.ds(c * cw, cw)

        def x_rdma(c):
            return pltpu.make_async_remote_copy(
                src_ref=pb.at[c % 2], dst_ref=xrv.at[c % 2],
                send_sem=x_send.at[c], recv_sem=x_recv.at[c],
                device_id=(1 - xi, yi, zi),
                device_id_type=pl.DeviceIdType.MESH,
            )

        def y_rdma(c):
            return pltpu.make_async_remote_copy(
                src_ref=mine.at[:, cols(c)], dst_ref=mine.at[:, cols(c)],
                send_sem=y_send.at[c], recv_sem=y_recv.at[c],
                device_id=(xi, 1 - yi, zi),
                device_id_type=pl.DeviceIdType.MESH,
            )

        def z_rdma(c):
            return pltpu.make_async_remote_copy(
                src_ref=mine.at[:, cols(c)], dst_ref=mine.at[:, cols(c)],
                send_sem=z_send.at[c], recv_sem=z_recv.at[c],
                device_id=(xi, yi, 1 - zi),
                device_id_type=pl.DeviceIdType.MESH,
            )

        cx = pltpu.make_async_copy(xs_ref, xs, copy_a)
        cx.start()
        pltpu.make_async_copy(dy_ref.at[:, cols(0)], dyb.at[0], copy_dy).start()
        cx.wait()

        def step(c, carry):
            slot = c % 2
            pltpu.make_async_copy(
                dy_ref.at[:, cols(c)], dyb.at[slot], copy_dy).wait()

            @pl.when(c + 1 < C)
            def _():
                pltpu.make_async_copy(
                    dy_ref.at[:, cols(c + 1)], dyb.at[1 - slot],
                    copy_dy).start()

            @pl.when(jnp.logical_and(jnp.logical_not(is_builder), c >= 2))
            def _():
                x_rdma(c - 2).wait_send()

            dybf[...] = dyb[slot].astype(jnp.bfloat16)
            pb[slot] = lax.dot_general(
                xs[...], dybf[...],
                (((1,), (0,)), ((), ())),
                preferred_element_type=jnp.float32,
            )

            @pl.when(jnp.logical_not(is_builder))
            def _():
                @pl.when(c >= 2)
                def _():
                    pl.semaphore_wait(x_credit, 1)
                x_rdma(c).start()
                @pl.when(c >= LAG)
                def _():
                    y_rdma(c - LAG).wait_recv()
                    z_rdma(c - LAG).start()

            @pl.when(is_builder)
            def _():
                x_rdma(c).wait_recv()
                vb[...] = pb[slot] + xrv[slot]
                pl.semaphore_signal(
                    x_credit, inc=1, device_id=(1 - xi, yi, zi),
                    device_id_type=pl.DeviceIdType.MESH,
                )
                co = pltpu.make_async_copy(vb, mine.at[:, cols(c)], copy_b)
                co.start()
                co.wait()
                y_rdma(c).start()
                z_rdma(c).start()

            return carry

        lax.fori_loop(0, C, step, 0)

        @pl.when(jnp.logical_not(is_builder))
        def _():
            def fwd(c, carry):
                y_rdma(c).wait_recv()
                z_rdma(c).start()
                return carry
            lax.fori_loop(C - LAG, C, fwd, 0)
            x_rdma(C - 2).wait_send()
            x_rdma(C - 1).wait_send()
            pl.semaphore_wait(x_credit, 2)

        @pl.when(is_builder)
        def _():
            def ws(c, carry):
                y_rdma(c).wait_send()
                return carry
            lax.fori_loop(0, C, ws, 0)

        def zd(c, carry):
            z_rdma(c).wait_send()
            z_rdma(c).wait_recv()
            return carry
        lax.fori_loop(0, C, zd, 0)

    return pl.pallas_call(
        body,
        out_shape=jax.ShapeDtypeStruct((d // 2, f), jnp.float32),
        in_specs=[
            pl.BlockSpec(memory_space=pl.ANY),
            pl.BlockSpec(memory_space=pl.ANY),
        ],
        out_specs=pl.BlockSpec(memory_space=pl.ANY),
        scratch_shapes=[
            pltpu.VMEM((blk, bl), jnp.bfloat16),
            pltpu.VMEM((2, bl, cw), jnp.float32),
            pltpu.VMEM((bl, cw), jnp.bfloat16),
            pltpu.VMEM((2, blk, cw), jnp.float32),
            pltpu.VMEM((blk, cw), jnp.float32),
            pltpu.VMEM((2, blk, cw), jnp.float32),
            pltpu.SemaphoreType.DMA((C,)),
            pltpu.SemaphoreType.DMA((C,)),
            pltpu.SemaphoreType.DMA((C,)),
            pltpu.SemaphoreType.DMA((C,)),
            pltpu.SemaphoreType.DMA((C,)),
            pltpu.SemaphoreType.DMA((C,)),
            pltpu.SemaphoreType.REGULAR,
            pltpu.SemaphoreType.DMA,
            pltpu.SemaphoreType.DMA,
            pltpu.SemaphoreType.DMA,
        ],
        compiler_params=pltpu.CompilerParams(
            collective_id=0, has_side_effects=True,
            vmem_limit_bytes=56 * 1024 * 1024,
        ),
    )(xt, dy)


# device time: 460826 ns/iter; 1.1027x vs baseline; 1.1027x over previous
import jax
import jax.numpy as jnp
from jax import lax
from jax.experimental import pallas as pl
from jax.experimental.pallas import tpu as pltpu

C = 32
LAG = 2


def kernel(x, dy):
    bl, d = x.shape
    _, f = dy.shape
    blk = d // 4
    cw = f // C

    xi = lax.axis_index("x")
    yi = lax.axis_index("y")
    zi = lax.axis_index("z")
    r = 2 * yi + zi

    x_slice = lax.dynamic_slice(x, (0, r * blk), (bl, blk))
    xt = x_slice.T.astype(jnp.bfloat16)

    def body(xs_ref, dy_ref, out_ref, xs, dyb, dybf, pb, vb, xrv,
             x_send, x_recv, y_send, y_recv, z_send, z_recv,
             x_credit, copy_a, copy_b, copy_dy):
        xi = lax.axis_index("x")
        yi = lax.axis_index("y")
        zi = lax.axis_index("z")
        is_builder = yi == xi

        barrier = pltpu.get_barrier_semaphore()
        for nbr in ((1 - xi, yi, zi), (xi, 1 - yi, zi), (xi, yi, 1 - zi)):
            pl.semaphore_signal(
                barrier, inc=1, device_id=nbr,
                device_id_type=pl.DeviceIdType.MESH,
            )
        pl.semaphore_wait(barrier, 3)

        mine = out_ref.at[pl.ds(zi * blk, blk)]
        stage = out_ref.at[pl.ds((1 - zi) * blk, blk)]

        def cols(c):
            return pl.ds(c * cw, cw)

        def x_rdma(c):
            return pltpu.make_async_remote_copy(
                src_ref=pb.at[c % 2], dst_ref=xrv.at[c % 2],
                send_sem=x_send.at[c], recv_sem=x_recv.at[c],
                device_id=(1 - xi, yi, zi),
                device_id_type=pl.DeviceIdType.MESH,
            )

        def y_rdma(c):
            return pltpu.make_async_remote_copy(
                src_ref=mine.at[:, cols(c)], dst_ref=mine.at[:, cols(c)],
                send_sem=y_send.at[c], recv_sem=y_recv.at[c],
                device_id=(xi, 1 - yi, zi),
                device_id_type=pl.DeviceIdType.MESH,
            )

        def z_rdma(c):
            return pltpu.make_async_remote_copy(
                src_ref=mine.at[:, cols(c)], dst_ref=mine.at[:, cols(c)],
                send_sem=z_send.at[c], recv_sem=z_recv.at[c],
                device_id=(xi, yi, 1 - zi),
                device_id_type=pl.DeviceIdType.MESH,
            )

        cx = pltpu.make_async_copy(xs_ref, xs, copy_a)
        cx.start()
        pltpu.make_async_copy(dy_ref.at[:, cols(0)], dyb.at[0], copy_dy).start()
        cx.wait()

        def step(c, carry):
            slot = c % 2
            pltpu.make_async_copy(
                dy_ref.at[:, cols(c)], dyb.at[slot], copy_dy).wait()

            @pl.when(c + 1 < C)
            def _():
                pltpu.make_async_copy(
                    dy_ref.at[:, cols(c + 1)], dyb.at[1 - slot],
                    copy_dy).start()

            @pl.when(jnp.logical_and(jnp.logical_not(is_builder), c >= 2))
            def _():
                x_rdma(c - 2).wait_send()

            dybf[...] = dyb[slot].astype(jnp.bfloat16)
            pb[slot] = lax.dot_general(
                xs[...], dybf[...],
                (((1,), (0,)), ((), ())),
                preferred_element_type=jnp.float32,
            )

            @pl.when(jnp.logical_not(is_builder))
            def _():
                @pl.when(c >= 2)
                def _():
                    pl.semaphore_wait(x_credit, 1)
                x_rdma(c).start()
                @pl.when(c >= LAG)
                def _():
                    y_rdma(c - LAG).wait_recv()
                    z_rdma(c - LAG).start()

            @pl.when(is_builder)
            def _():
                x_rdma(c).wait_recv()
                vb[...] = pb[slot] + xrv[slot]
                pl.semaphore_signal(
                    x_credit, inc=1, device_id=(1 - xi, yi, zi),
                    device_id_type=pl.DeviceIdType.MESH,
                )
                co = pltpu.make_async_copy(vb, mine.at[:, cols(c)], copy_b)
                co.start()
                co.wait()
                y_rdma(c).start()
                z_rdma(c).start()

            return carry

        lax.fori_loop(0, C, step, 0)

        @pl.when(jnp.logical_not(is_builder))
        def _():
            def fwd(c, carry):
                y_rdma(c).wait_recv()
                z_rdma(c).start()
                return carry
            lax.fori_loop(C - LAG, C, fwd, 0)
            x_rdma(C - 2).wait_send()
            x_rdma(C - 1).wait_send()
            pl.semaphore_wait(x_credit, 2)

        @pl.when(is_builder)
        def _():
            def ws(c, carry):
                y_rdma(c).wait_send()
                return carry
            lax.fori_loop(0, C, ws, 0)

        def zd(c, carry):
            z_rdma(c).wait_send()
            z_rdma(c).wait_recv()
            return carry
        lax.fori_loop(0, C, zd, 0)

    return pl.pallas_call(
        body,
        out_shape=jax.ShapeDtypeStruct((d // 2, f), jnp.float32),
        in_specs=[
            pl.BlockSpec(memory_space=pl.ANY),
            pl.BlockSpec(memory_space=pl.ANY),
        ],
        out_specs=pl.BlockSpec(memory_space=pl.ANY),
        scratch_shapes=[
            pltpu.VMEM((blk, bl), jnp.bfloat16),
            pltpu.VMEM((2, bl, cw), jnp.float32),
            pltpu.VMEM((bl, cw), jnp.bfloat16),
            pltpu.VMEM((2, blk, cw), jnp.float32),
            pltpu.VMEM((blk, cw), jnp.float32),
            pltpu.VMEM((2, blk, cw), jnp.float32),
            pltpu.SemaphoreType.DMA((C,)),
            pltpu.SemaphoreType.DMA((C,)),
            pltpu.SemaphoreType.DMA((C,)),
            pltpu.SemaphoreType.DMA((C,)),
            pltpu.SemaphoreType.DMA((C,)),
            pltpu.SemaphoreType.DMA((C,)),
            pltpu.SemaphoreType.REGULAR,
            pltpu.SemaphoreType.DMA,
            pltpu.SemaphoreType.DMA,
            pltpu.SemaphoreType.DMA,
        ],
        compiler_params=pltpu.CompilerParams(
            collective_id=0, has_side_effects=True,
            vmem_limit_bytes=56 * 1024 * 1024,
        ),
    )(xt, dy)


# device time: 459848 ns/iter; 1.1050x vs baseline; 1.0021x over previous
import jax
import jax.numpy as jnp
from jax import lax
from jax.experimental import pallas as pl
from jax.experimental.pallas import tpu as pltpu

C = 32
LAG = 3


def kernel(x, dy):
    bl, d = x.shape
    _, f = dy.shape
    blk = d // 4
    cw = f // C

    xi = lax.axis_index("x")
    yi = lax.axis_index("y")
    zi = lax.axis_index("z")
    r = 2 * yi + zi

    x_slice = lax.dynamic_slice(x, (0, r * blk), (bl, blk))
    xt = x_slice.T.astype(jnp.bfloat16)

    def body(xs_ref, dy_ref, out_ref, xs, dyb, dybf, pb, vb, xrv,
             x_send, x_recv, y_send, y_recv, z_send, z_recv,
             x_credit, copy_a, copy_b, copy_dy):
        xi = lax.axis_index("x")
        yi = lax.axis_index("y")
        zi = lax.axis_index("z")
        is_builder = yi == xi

        barrier = pltpu.get_barrier_semaphore()
        for nbr in ((1 - xi, yi, zi), (xi, 1 - yi, zi), (xi, yi, 1 - zi)):
            pl.semaphore_signal(
                barrier, inc=1, device_id=nbr,
                device_id_type=pl.DeviceIdType.MESH,
            )
        pl.semaphore_wait(barrier, 3)

        mine = out_ref.at[pl.ds(zi * blk, blk)]
        stage = out_ref.at[pl.ds((1 - zi) * blk, blk)]

        def cols(c):
            return pl.ds(c * cw, cw)

        def x_rdma(c):
            return pltpu.make_async_remote_copy(
                src_ref=pb.at[c % 2], dst_ref=xrv.at[c % 2],
                send_sem=x_send.at[c], recv_sem=x_recv.at[c],
                device_id=(1 - xi, yi, zi),
                device_id_type=pl.DeviceIdType.MESH,
            )

        def y_rdma(c):
            return pltpu.make_async_remote_copy(
                src_ref=mine.at[:, cols(c)], dst_ref=mine.at[:, cols(c)],
                send_sem=y_send.at[c], recv_sem=y_recv.at[c],
                device_id=(xi, 1 - yi, zi),
                device_id_type=pl.DeviceIdType.MESH,
            )

        def z_rdma(c):
            return pltpu.make_async_remote_copy(
                src_ref=mine.at[:, cols(c)], dst_ref=mine.at[:, cols(c)],
                send_sem=z_send.at[c], recv_sem=z_recv.at[c],
                device_id=(xi, yi, 1 - zi),
                device_id_type=pl.DeviceIdType.MESH,
            )

        cx = pltpu.make_async_copy(xs_ref, xs, copy_a)
        cx.start()
        pltpu.make_async_copy(dy_ref.at[:, cols(0)], dyb.at[0], copy_dy).start()
        cx.wait()

        def step(c, carry):
            slot = c % 2
            pltpu.make_async_copy(
                dy_ref.at[:, cols(c)], dyb.at[slot], copy_dy).wait()

            @pl.when(c + 1 < C)
            def _():
                pltpu.make_async_copy(
                    dy_ref.at[:, cols(c + 1)], dyb.at[1 - slot],
                    copy_dy).start()

            @pl.when(jnp.logical_and(jnp.logical_not(is_builder), c >= 2))
            def _():
                x_rdma(c - 2).wait_send()

            dybf[...] = dyb[slot].astype(jnp.bfloat16)
            pb[slot] = lax.dot_general(
                xs[...], dybf[...],
                (((1,), (0,)), ((), ())),
                preferred_element_type=jnp.float32,
            )

            @pl.when(jnp.logical_not(is_builder))
            def _():
                @pl.when(c >= 2)
                def _():
                    pl.semaphore_wait(x_credit, 1)
                x_rdma(c).start()
                @pl.when(c >= LAG)
                def _():
                    y_rdma(c - LAG).wait_recv()
                    z_rdma(c - LAG).start()

            @pl.when(is_builder)
            def _():
                x_rdma(c).wait_recv()
                vb[...] = pb[slot] + xrv[slot]
                pl.semaphore_signal(
                    x_credit, inc=1, device_id=(1 - xi, yi, zi),
                    device_id_type=pl.DeviceIdType.MESH,
                )
                co = pltpu.make_async_copy(vb, mine.at[:, cols(c)], copy_b)
                co.start()
                co.wait()
                y_rdma(c).start()
                z_rdma(c).start()

            return carry

        lax.fori_loop(0, C, step, 0)

        @pl.when(jnp.logical_not(is_builder))
        def _():
            def fwd(c, carry):
                y_rdma(c).wait_recv()
                z_rdma(c).start()
                return carry
            lax.fori_loop(C - LAG, C, fwd, 0)
            x_rdma(C - 2).wait_send()
            x_rdma(C - 1).wait_send()
            pl.semaphore_wait(x_credit, 2)

        @pl.when(is_builder)
        def _():
            def ws(c, carry):
                y_rdma(c).wait_send()
                return carry
            lax.fori_loop(0, C, ws, 0)

        def zd(c, carry):
            z_rdma(c).wait_send()
            z_rdma(c).wait_recv()
            return carry
        lax.fori_loop(0, C, zd, 0)

    return pl.pallas_call(
        body,
        out_shape=jax.ShapeDtypeStruct((d // 2, f), jnp.float32),
        in_specs=[
            pl.BlockSpec(memory_space=pl.ANY),
            pl.BlockSpec(memory_space=pl.ANY),
        ],
        out_specs=pl.BlockSpec(memory_space=pl.ANY),
        scratch_shapes=[
            pltpu.VMEM((blk, bl), jnp.bfloat16),
            pltpu.VMEM((2, bl, cw), jnp.float32),
            pltpu.VMEM((bl, cw), jnp.bfloat16),
            pltpu.VMEM((2, blk, cw), jnp.float32),
            pltpu.VMEM((blk, cw), jnp.float32),
            pltpu.VMEM((2, blk, cw), jnp.float32),
            pltpu.SemaphoreType.DMA((C,)),
            pltpu.SemaphoreType.DMA((C,)),
            pltpu.SemaphoreType.DMA((C,)),
            pltpu.SemaphoreType.DMA((C,)),
            pltpu.SemaphoreType.DMA((C,)),
            pltpu.SemaphoreType.DMA((C,)),
            pltpu.SemaphoreType.REGULAR,
            pltpu.SemaphoreType.DMA,
            pltpu.SemaphoreType.DMA,
            pltpu.SemaphoreType.DMA,
        ],
        compiler_params=pltpu.CompilerParams(
            collective_id=0, has_side_effects=True,
            vmem_limit_bytes=56 * 1024 * 1024,
        ),
    )(xt, dy)
